# baseline (speedup 1.0000x reference)
import functools

import jax
import jax.numpy as jnp
from jax import lax
from jax.experimental import pallas as pl
from jax.experimental.pallas import tpu as pltpu
from jax.experimental.pallas import tpu_sc as plsc

D = 128
DW = D // 2
LANES = 16
NW = 32
C = 80


def _pack_bf16(a):
    n = a.shape[0]
    return lax.bitcast_convert_type(
        a.astype(jnp.bfloat16).reshape(n, DW, 2), jnp.float32)


def _sc_decode(zp, src_idx, dst_idx, rel_idx, wp, n_edges):
    epw = n_edges // NW
    n_chunks = epw // C

    mesh = plsc.VectorSubcoreMesh(core_axis_name="c", subcore_axis_name="s")

    @functools.partial(
        pl.kernel,
        out_type=jax.ShapeDtypeStruct((n_edges,), jnp.float32),
        mesh=mesh,
        compiler_params=pltpu.CompilerParams(needs_layout_passes=False,
                                             use_tc_tiling_on_sc=False),
        scratch_types=[
            pltpu.VMEM((epw,), jnp.int32),
            pltpu.VMEM((epw,), jnp.int32),
            pltpu.VMEM((epw,), jnp.int32),
            pltpu.VMEM((C, DW), jnp.float32),
            pltpu.VMEM((C, DW), jnp.float32),
            pltpu.VMEM((C, DW), jnp.float32),
            pltpu.VMEM((C, DW), jnp.float32),
            pltpu.VMEM((C, DW), jnp.float32),
            pltpu.VMEM((C, DW), jnp.float32),
            pltpu.VMEM((wp.shape[0], DW), jnp.float32),
            pltpu.VMEM((LANES, LANES), jnp.float32),
            pltpu.VMEM((C,), jnp.float32),
            pltpu.VMEM((C,), jnp.float32),
            pltpu.VMEM((C,), jnp.float32),
            pltpu.SemaphoreType.DMA,
            pltpu.SemaphoreType.DMA,
            pltpu.SemaphoreType.DMA,
            pltpu.SemaphoreType.DMA,
        ],
    )
    def decode(z_hbm, src_hbm, dst_hbm, rel_hbm, w_hbm, out_hbm,
               si_v, di_v, ri_v, sr0, dr0, sr1, dr1,
               sr2, dr2, w_t, t_v, ob0, ob1, ob2, sem0, sem1, sem2, sem_o):
        wid = lax.axis_index("s") * 2 + lax.axis_index("c")
        base0 = wid * epw
        iota = lax.iota(jnp.int32, LANES)

        pltpu.sync_copy(w_hbm, w_t)
        pltpu.sync_copy(src_hbm.at[pl.ds(base0, epw)], si_v)
        pltpu.sync_copy(dst_hbm.at[pl.ds(base0, epw)], di_v)
        pltpu.sync_copy(rel_hbm.at[pl.ds(base0, epw)], ri_v)

        def row_copies(g, sr, dr, sem):
            off = g * C
            return (
                pltpu.make_async_copy(z_hbm.at[si_v.at[pl.ds(off, C)]], sr, sem),
                pltpu.make_async_copy(z_hbm.at[di_v.at[pl.ds(off, C)]], dr, sem),
            )

        def issue(g, sr, dr, sem):
            for cp in row_copies(g, sr, dr, sem):
                cp.start()

        def wait(g, sr, dr, sem):
            for cp in row_copies(g, sr, dr, sem):
                cp.wait()

        def edge_loads(sr, dr, gb, k, rel):
            return [(sr[gb + k, pl.ds(j * LANES, LANES)],
                     dr[gb + k, pl.ds(j * LANES, LANES)],
                     w_t[rel, pl.ds(j * LANES, LANES)])
                    for j in range(DW // LANES)]

        def edge_score(loaded):
            halves = []
            for sv, dv, wv in loaded:
                p = (plsc.bitcast(sv, jnp.bfloat16)
                     * plsc.bitcast(dv, jnp.bfloat16)
                     * plsc.bitcast(wv, jnp.bfloat16))
                p0, p1 = plsc.unpack(p, format=plsc.PackFormat.INTERLEAVED)
                halves += [p0, p1]
            while len(halves) > 1:
                halves = [a + b for a, b in zip(halves[0::2], halves[1::2])]
            return halves[0]

        def out_copy(g, ob):
            return pltpu.make_async_copy(
                ob, out_hbm.at[pl.ds(base0 + g * C, C)], sem_o)

        def score_chunk(g, sr, dr, ob):
            def group_body(grp, carry):
                gb = grp * LANES
                relv = ri_v[pl.ds(g * C + gb, LANES)]

                cur = edge_loads(sr, dr, gb, 0, relv[0])
                for k in range(LANES):
                    nxt = (edge_loads(sr, dr, gb, k + 1, relv[k + 1])
                           if k + 1 < LANES else None)
                    t_v[k, :] = edge_score(cur)
                    cur = nxt

                cols = [plsc.load_gather(
                            t_v, [iota, jnp.full((LANES,), k, jnp.int32)])
                        for k in range(LANES)]
                while len(cols) > 1:
                    cols = [a + b for a, b in zip(cols[0::2], cols[1::2])]
                s = 1.0 / (1.0 + jnp.exp(-cols[0]))
                ob[pl.ds(gb, LANES)] = s
                return carry

            lax.fori_loop(0, C // LANES, group_body, 0)
            out_copy(g, ob).start()

        bufs = ((sr0, dr0, sem0), (sr1, dr1, sem1), (sr2, dr2, sem2))
        obs = (ob0, ob1, ob2)
        issue(0, *bufs[0])
        issue(1, *bufs[1])

        def triple_body(i, carry):
            g = 3 * i
            for p in range(3):
                gc = g + p
                issue(gc + 2, *bufs[(p + 2) % 3])
                wait(gc, *bufs[p])

                @pl.when(gc >= 3)
                def _():
                    out_copy(gc - 3, obs[p]).wait()

                score_chunk(gc, *bufs[p][:2], obs[p])
            return carry

        lax.fori_loop(0, (n_chunks - 2) // 3, triple_body, 0)
        for g in range(n_chunks - 2, n_chunks):
            wait(g, *bufs[g % 3])
            out_copy(g - 3, obs[g % 3]).wait()
            score_chunk(g, *bufs[g % 3][:2], obs[g % 3])
        for g in range(n_chunks - 3, n_chunks):
            out_copy(g, obs[g % 3]).wait()

    return decode(zp, src_idx, dst_idx, rel_idx, wp)


def kernel(z, edge_index, edge_type, weight):
    n_edges = edge_index.shape[1]
    src_idx = edge_index[0]
    dst_idx = edge_index[1]
    return _sc_decode(_pack_bf16(z), src_idx, dst_idx, edge_type,
                      _pack_bf16(weight), n_edges)
